# baseline (speedup 1.0000x reference)
import functools

import jax
import jax.numpy as jnp
from jax import lax
from jax.experimental import pallas as pl
from jax.experimental.pallas import tpu as pltpu
from jax.experimental.pallas import tpu_sc as plsc

_N = 10000
_E = 320000
_D = 128
_H = 128

_NC = 2
_NS = 16
_L = 16
_NW = _NC * _NS
_EPW = _E // _NW
_CH = 80
_NCHUNK = _EPW // _CH
_G = 1
_GB = _CH // _G
_RPT = 624
_ZLEN = 2000


def _sc_agg_body(h2, srcb, dstb, efb, agg_out, den_out0, den_out1,
                 src_v, dst_v, sig_v, rows0, rows1, zv, agg_sh, den_sh,
                 gsem, ssem, isem0, isem1):
  c = lax.axis_index("c")
  s = lax.axis_index("s")
  wid = s * _NC + c

  zero16 = jnp.zeros((_L,), jnp.float32)

  def _zr(i, carry):
    for k in range(_D // _L):
      rows0[0, i, pl.ds(k * _L, _L)] = zero16
    return carry
  lax.fori_loop(0, _CH, _zr, 0)

  def _zz(i, carry):
    zv[pl.ds(i * _L, _L)] = zero16
    return carry
  lax.fori_loop(0, _ZLEN // _L, _zz, 0)

  for z in range(_RPT // _CH):
    pltpu.sync_copy(rows0.at[0], agg_sh.at[pl.ds(s * _RPT + z * _CH, _CH)])
  pltpu.sync_copy(rows0.at[0].at[pl.ds(0, _RPT % _CH)],
                  agg_sh.at[pl.ds(s * _RPT + (_RPT // _CH) * _CH,
                                  _RPT % _CH)])

  @pl.when(s == _NS - 1)
  def _():
    pltpu.sync_copy(rows0.at[0].at[pl.ds(0, _N - _NS * _RPT)],
                    agg_sh.at[pl.ds(_NS * _RPT, _N - _NS * _RPT)])

  @pl.when(s == 0)
  def _():
    for i in range(_N // _ZLEN):
      pltpu.sync_copy(zv, den_sh.at[pl.ds(i * _ZLEN, _ZLEN)])

  plsc.subcore_barrier()

  ebase0 = wid * _EPW

  def _stage_fire(slot, ch, sem):
    eb = ebase0 + ch * _CH
    pltpu.async_copy(srcb.at[pl.ds(eb, _CH)], src_v.at[slot], sem)
    pltpu.async_copy(dstb.at[pl.ds(eb, _CH)], dst_v.at[slot], sem)
    pltpu.async_copy(efb.at[pl.ds(eb, _CH)], sig_v.at[slot], sem)

  def _stage_drain(sem):
    pltpu.make_async_copy(srcb.at[pl.ds(0, _CH)], src_v.at[0], sem).wait()
    pltpu.make_async_copy(dstb.at[pl.ds(0, _CH)], dst_v.at[0], sem).wait()
    pltpu.make_async_copy(efb.at[pl.ds(0, _CH)], sig_v.at[0], sem).wait()

  def _gather_fire(par, slot):
    pltpu.async_copy(h2.at[dst_v.at[slot]], rows0.at[par], gsem)
    pltpu.async_copy(h2.at[src_v.at[slot]], rows1.at[par], gsem)

  def _gather_drain():
    pltpu.make_async_copy(h2.at[pl.ds(0, _CH)], rows0.at[0], gsem).wait()
    pltpu.make_async_copy(h2.at[pl.ds(0, _CH)], rows1.at[0], gsem).wait()

  def _scatter_fire(par, slot):
    pltpu.async_copy(rows0.at[par], agg_sh.at[src_v.at[slot]], ssem,
                     add=True)
    pltpu.async_copy(rows1.at[par], agg_sh.at[dst_v.at[slot]], ssem,
                     add=True)
    pltpu.async_copy(sig_v.at[slot], den_sh.at[src_v.at[slot]], ssem,
                     add=True)
    pltpu.async_copy(sig_v.at[slot], den_sh.at[dst_v.at[slot]], ssem,
                     add=True)

  def _scatter_drain():
    pltpu.make_async_copy(rows0.at[0], agg_sh.at[pl.ds(0, _CH)],
                          ssem).wait()
    pltpu.make_async_copy(rows1.at[0], agg_sh.at[pl.ds(0, _CH)],
                          ssem).wait()
    pltpu.make_async_copy(sig_v.at[0], den_sh.at[pl.ds(0, _CH)],
                          ssem).wait()
    pltpu.make_async_copy(sig_v.at[0], den_sh.at[pl.ds(0, _CH)],
                          ssem).wait()

  bdnums = lax.GatherDimensionNumbers(
      offset_dims=(), collapsed_slice_dims=(0,), start_index_map=(0,))

  def _process(par, slot):
    for t in range(_CH // _L):
      sl = (slot, pl.ds(t * _L, _L))
      e = sig_v[sl]
      sig_v[sl] = 1.0 / (1.0 + jnp.exp(-e))

    def _srow(row, carry2):
      t = lax.shift_right_logical(row, 4)
      r = jnp.bitwise_and(row, _L - 1)
      sv = sig_v[slot, pl.ds(t * _L, _L)]
      b = lax.gather(sv, jnp.full((_L, 1), r, jnp.int32), bdnums, (1,),
                     mode=lax.GatherScatterMode.PROMISE_IN_BOUNDS)
      for kk in range(_D // _L):
        sl = (par, row, pl.ds(kk * _L, _L))
        rows0[sl] = rows0[sl] * b
        rows1[sl] = rows1[sl] * b
      return carry2
    lax.fori_loop(0, _CH, _srow, 0)

  pltpu.sync_copy(srcb.at[pl.ds(ebase0, _CH)], src_v.at[0])
  pltpu.sync_copy(dstb.at[pl.ds(ebase0, _CH)], dst_v.at[0])
  pltpu.sync_copy(efb.at[pl.ds(ebase0, _CH)], sig_v.at[0])
  _stage_fire(1, 1, isem0)
  _gather_fire(0, 0)

  _gather_drain()
  _stage_drain(isem0)
  _gather_fire(1, 1)
  _stage_fire(2, 2, isem0)
  _process(0, 0)
  _scatter_fire(0, 0)

  def _chunk(k, carry):
    par = lax.rem(k, 2)
    par1 = lax.rem(k + 1, 2)
    slot = lax.rem(k, 3)
    slot1 = lax.rem(k + 1, 3)
    slot2 = lax.rem(k + 2, 3)
    with jax.named_scope("wait_gather"):
      _gather_drain()
    with jax.named_scope("wait_scatter"):
      _scatter_drain()
    with jax.named_scope("wait_idx"):
      _stage_drain(isem0)
    with jax.named_scope("fires"):
      _gather_fire(par1, slot1)
      _stage_fire(slot2, k + 2, isem0)
    with jax.named_scope("process"):
      _process(par, slot)
    with jax.named_scope("fire_scatter"):
      _scatter_fire(par, slot)
    return carry

  lax.fori_loop(1, _NCHUNK - 2, _chunk, 0)

  kA = _NCHUNK - 2
  _gather_drain()
  _scatter_drain()
  _stage_drain(isem0)
  _gather_fire((kA + 1) % 2, (kA + 1) % 3)
  _process(kA % 2, kA % 3)
  _scatter_fire(kA % 2, kA % 3)

  kB = _NCHUNK - 1
  _gather_drain()
  _scatter_drain()
  _process(kB % 2, kB % 3)
  _scatter_fire(kB % 2, kB % 3)
  _scatter_drain()

  plsc.subcore_barrier()

  pltpu.sync_copy(agg_sh.at[pl.ds(s * _RPT, _RPT)],
                  agg_out.at[c].at[pl.ds(s * _RPT, _RPT)])

  @pl.when(s == _NS - 1)
  def _():
    pltpu.sync_copy(agg_sh.at[pl.ds(_NS * _RPT, _N - _NS * _RPT)],
                    agg_out.at[c].at[pl.ds(_NS * _RPT, _N - _NS * _RPT)])

  @pl.when(jnp.logical_and(s == 0, c == 0))
  def _():
    pltpu.sync_copy(den_sh, den_out0)

  @pl.when(jnp.logical_and(s == 0, c == 1))
  def _():
    pltpu.sync_copy(den_sh, den_out1)


def _sc_agg(h2, src, dst, ef):
  mesh = plsc.VectorSubcoreMesh(
      core_axis_name="c", subcore_axis_name="s",
      num_cores=_NC, num_subcores=_NS)
  fn = pl.kernel(
      _sc_agg_body,
      out_type=[
          jax.ShapeDtypeStruct((_NC, _N, _D), jnp.float32),
          jax.ShapeDtypeStruct((_N,), jnp.float32),
          jax.ShapeDtypeStruct((_N,), jnp.float32),
      ],
      mesh=mesh,
      scratch_types=[
          pltpu.VMEM((3, _CH), jnp.int32),
          pltpu.VMEM((3, _CH), jnp.int32),
          pltpu.VMEM((3, _CH), jnp.float32),
          pltpu.VMEM((2, _CH, _D), jnp.float32),
          pltpu.VMEM((2, _CH, _D), jnp.float32),
          pltpu.VMEM((_ZLEN,), jnp.float32),
          pltpu.VMEM_SHARED((_N, _D), jnp.float32),
          pltpu.VMEM_SHARED((_N,), jnp.float32),
          pltpu.SemaphoreType.DMA,
          pltpu.SemaphoreType.DMA,
          pltpu.SemaphoreType.DMA,
          pltpu.SemaphoreType.DMA,
      ],
      name="sc_edge_aggregate",
  )
  return fn(h2, src, dst, ef)


def _tc_pre_body(x_ref, w1_ref, b1_ref, w2_ref, b2_ref, h2_ref):
  x = x_ref[...]
  h = lax.dot_general(x, w1_ref[...], (((1,), (1,)), ((), ())),
                      preferred_element_type=jnp.float32)
  h = jnp.maximum(h + b1_ref[...], 0.0)
  h2_ref[...] = lax.dot_general(h, w2_ref[...], (((1,), (1,)), ((), ())),
                                preferred_element_type=jnp.float32) + b2_ref[...]


_RB = 2000


def _tc_pre(x, w1, b1, w2, b2):
  nb = _N // _RB
  return pl.pallas_call(
      _tc_pre_body,
      grid=(nb,),
      in_specs=[
          pl.BlockSpec((_RB, _D), lambda i: (i, 0)),
          pl.BlockSpec((_H, _D), lambda i: (0, 0)),
          pl.BlockSpec((1, _H), lambda i: (0, 0)),
          pl.BlockSpec((_D, _H), lambda i: (0, 0)),
          pl.BlockSpec((1, _D), lambda i: (0, 0)),
      ],
      out_specs=pl.BlockSpec((_RB, _D), lambda i: (i, 0)),
      out_shape=jax.ShapeDtypeStruct((_N, _D), jnp.float32),
  )(x, w1, b1.reshape(1, _H), w2, b2.reshape(1, _D))


def _tc_post_body(x_ref, w1_ref, b1_ref, w2_ref, b2_ref, agg_ref, den0_ref,
                  den1_ref, o_ref):
  x = x_ref[...]
  h = lax.dot_general(x, w1_ref[...], (((1,), (1,)), ((), ())),
                      preferred_element_type=jnp.float32)
  h = jnp.maximum(h + b1_ref[...], 0.0)
  h1 = lax.dot_general(h, w2_ref[...], (((1,), (1,)), ((), ())),
                       preferred_element_type=jnp.float32) + b2_ref[...]
  agg = agg_ref[0] + agg_ref[1]
  den = den0_ref[...] + den1_ref[...] + 1e-07
  inter = h1 + agg / den
  mean = jnp.mean(inter, axis=1, keepdims=True)
  cen = inter - mean
  var = jnp.mean(cen * cen, axis=1, keepdims=True)
  normed = cen * lax.rsqrt(var + 1e-05)
  o_ref[...] = x + jnp.maximum(normed, 0.0)


def _tc_post(x, w1, b1, w2, b2, agg_p, den0, den1):
  nb = _N // _RB
  return pl.pallas_call(
      _tc_post_body,
      grid=(nb,),
      in_specs=[
          pl.BlockSpec((_RB, _D), lambda i: (i, 0)),
          pl.BlockSpec((_H, _D), lambda i: (0, 0)),
          pl.BlockSpec((1, _H), lambda i: (0, 0)),
          pl.BlockSpec((_D, _H), lambda i: (0, 0)),
          pl.BlockSpec((1, _D), lambda i: (0, 0)),
          pl.BlockSpec((_NC, _RB, _D), lambda i: (0, i, 0)),
          pl.BlockSpec((_RB, 1), lambda i: (i, 0)),
          pl.BlockSpec((_RB, 1), lambda i: (i, 0)),
      ],
      out_specs=pl.BlockSpec((_RB, _D), lambda i: (i, 0)),
      out_shape=jax.ShapeDtypeStruct((_N, _D), jnp.float32),
  )(x, w1, b1.reshape(1, _H), w2, b2.reshape(1, _D), agg_p,
    den0.reshape(_N, 1), den1.reshape(_N, 1))


def kernel(node_features, edge_index, edge_features,
           W1a, b1a, W2a, b2a, W1b, b1b, W2b, b2b):
  src = edge_index[0].astype(jnp.int32)
  dst = edge_index[1].astype(jnp.int32)
  h2 = _tc_pre(node_features, W1b, b1b, W2b, b2b)
  agg_p, den0, den1 = _sc_agg(h2, src, dst, edge_features)
  return _tc_post(node_features, W1a, b1a, W2a, b2a, agg_p, den0, den1)


# speedup vs baseline: 2.7841x; 2.7841x over previous
import functools

import jax
import jax.numpy as jnp
from jax import lax
from jax.experimental import pallas as pl
from jax.experimental.pallas import tpu as pltpu
from jax.experimental.pallas import tpu_sc as plsc

_N = 10000
_E = 320000
_D = 128
_H = 128

_NC = 2
_NS = 16
_L = 16
_NW = _NC * _NS
_EPW = _E // _NW
_CH = 80
_NCHUNK = _EPW // _CH
_G = 1
_GB = _CH // _G
_RPT = 624
_ZLEN = 2000


def _sc_agg_body(h2, srcb, dstb, efb, agg_out, den_out0, den_out1,
                 src_v, dst_v, sig_v, rows0, rows1, zv, agg_sh, den_sh,
                 gsem, ssem, isem0, isem1):
  c = lax.axis_index("c")
  s = lax.axis_index("s")
  wid = s * _NC + c

  zero16 = jnp.zeros((_L,), jnp.float32)

  def _zr(i, carry):
    for k in range(_D // _L):
      rows0[0, i, pl.ds(k * _L, _L)] = zero16
    return carry
  lax.fori_loop(0, _CH, _zr, 0)

  def _zz(i, carry):
    zv[pl.ds(i * _L, _L)] = zero16
    return carry
  lax.fori_loop(0, _ZLEN // _L, _zz, 0)

  for z in range(_RPT // _CH):
    pltpu.sync_copy(rows0.at[0], agg_sh.at[pl.ds(s * _RPT + z * _CH, _CH)])
  pltpu.sync_copy(rows0.at[0].at[pl.ds(0, _RPT % _CH)],
                  agg_sh.at[pl.ds(s * _RPT + (_RPT // _CH) * _CH,
                                  _RPT % _CH)])

  @pl.when(s == _NS - 1)
  def _():
    pltpu.sync_copy(rows0.at[0].at[pl.ds(0, _N - _NS * _RPT)],
                    agg_sh.at[pl.ds(_NS * _RPT, _N - _NS * _RPT)])

  @pl.when(s == 0)
  def _():
    for i in range(_N // _ZLEN):
      pltpu.sync_copy(zv, den_sh.at[pl.ds(i * _ZLEN, _ZLEN)])

  plsc.subcore_barrier()

  ebase0 = wid * _EPW

  def _stage_fire(slot, ch, sem):
    eb = ebase0 + ch * _CH
    pltpu.async_copy(srcb.at[pl.ds(eb, _CH)], src_v.at[slot], sem)
    pltpu.async_copy(dstb.at[pl.ds(eb, _CH)], dst_v.at[slot], sem)
    pltpu.async_copy(efb.at[pl.ds(eb, _CH)], sig_v.at[slot], sem)

  def _stage_drain(sem):
    pltpu.make_async_copy(srcb.at[pl.ds(0, _CH)], src_v.at[0], sem).wait()
    pltpu.make_async_copy(dstb.at[pl.ds(0, _CH)], dst_v.at[0], sem).wait()
    pltpu.make_async_copy(efb.at[pl.ds(0, _CH)], sig_v.at[0], sem).wait()

  def _gather_fire(par, slot):
    pltpu.async_copy(h2.at[dst_v.at[slot]], rows0.at[par], gsem)
    pltpu.async_copy(h2.at[src_v.at[slot]], rows1.at[par], gsem)

  def _gather_drain():
    pltpu.make_async_copy(h2.at[pl.ds(0, _CH)], rows0.at[0], gsem).wait()
    pltpu.make_async_copy(h2.at[pl.ds(0, _CH)], rows1.at[0], gsem).wait()

  def _scatter_fire(par, slot):
    pltpu.async_copy(rows0.at[par], agg_sh.at[src_v.at[slot]], ssem,
                     add=True)
    pltpu.async_copy(rows1.at[par], agg_sh.at[dst_v.at[slot]], ssem,
                     add=True)
    pltpu.async_copy(sig_v.at[slot], den_sh.at[src_v.at[slot]], ssem,
                     add=True)
    pltpu.async_copy(sig_v.at[slot], den_sh.at[dst_v.at[slot]], ssem,
                     add=True)

  def _scatter_drain():
    pltpu.make_async_copy(rows0.at[0], agg_sh.at[pl.ds(0, _CH)],
                          ssem).wait()
    pltpu.make_async_copy(rows1.at[0], agg_sh.at[pl.ds(0, _CH)],
                          ssem).wait()
    pltpu.make_async_copy(sig_v.at[0], den_sh.at[pl.ds(0, _CH)],
                          ssem).wait()
    pltpu.make_async_copy(sig_v.at[0], den_sh.at[pl.ds(0, _CH)],
                          ssem).wait()

  bdnums = lax.GatherDimensionNumbers(
      offset_dims=(), collapsed_slice_dims=(0,), start_index_map=(0,))

  def _process(par, slot):
    for t in range(_CH // _L):
      sl = (slot, pl.ds(t * _L, _L))
      e = sig_v[sl]
      sig_v[sl] = 1.0 / (1.0 + jnp.exp(-e))

    @plsc.parallel_loop(0, _CH, 1, unroll=4)
    def _srow(row):
      t = lax.shift_right_logical(row, 4)
      r = jnp.bitwise_and(row, _L - 1)
      sv = sig_v[slot, pl.ds(t * _L, _L)]
      b = lax.gather(sv, jnp.full((_L, 1), r, jnp.int32), bdnums, (1,),
                     mode=lax.GatherScatterMode.PROMISE_IN_BOUNDS)
      for kk in range(_D // _L):
        sl = (par, row, pl.ds(kk * _L, _L))
        rows0[sl] = rows0[sl] * b
        rows1[sl] = rows1[sl] * b

  pltpu.sync_copy(srcb.at[pl.ds(ebase0, _CH)], src_v.at[0])
  pltpu.sync_copy(dstb.at[pl.ds(ebase0, _CH)], dst_v.at[0])
  pltpu.sync_copy(efb.at[pl.ds(ebase0, _CH)], sig_v.at[0])
  _stage_fire(1, 1, isem0)
  _gather_fire(0, 0)

  _gather_drain()
  _stage_drain(isem0)
  _gather_fire(1, 1)
  _stage_fire(2, 2, isem0)
  _process(0, 0)
  _scatter_fire(0, 0)

  def _chunk(k, carry):
    par = lax.rem(k, 2)
    par1 = lax.rem(k + 1, 2)
    slot = lax.rem(k, 3)
    slot1 = lax.rem(k + 1, 3)
    slot2 = lax.rem(k + 2, 3)
    with jax.named_scope("wait_gather"):
      _gather_drain()
    with jax.named_scope("wait_scatter"):
      _scatter_drain()
    with jax.named_scope("wait_idx"):
      _stage_drain(isem0)
    with jax.named_scope("fires"):
      _gather_fire(par1, slot1)
      _stage_fire(slot2, k + 2, isem0)
    with jax.named_scope("process"):
      _process(par, slot)
    with jax.named_scope("fire_scatter"):
      _scatter_fire(par, slot)
    return carry

  lax.fori_loop(1, _NCHUNK - 2, _chunk, 0)

  kA = _NCHUNK - 2
  _gather_drain()
  _scatter_drain()
  _stage_drain(isem0)
  _gather_fire((kA + 1) % 2, (kA + 1) % 3)
  _process(kA % 2, kA % 3)
  _scatter_fire(kA % 2, kA % 3)

  kB = _NCHUNK - 1
  _gather_drain()
  _scatter_drain()
  _process(kB % 2, kB % 3)
  _scatter_fire(kB % 2, kB % 3)
  _scatter_drain()

  plsc.subcore_barrier()

  pltpu.sync_copy(agg_sh.at[pl.ds(s * _RPT, _RPT)],
                  agg_out.at[c].at[pl.ds(s * _RPT, _RPT)])

  @pl.when(s == _NS - 1)
  def _():
    pltpu.sync_copy(agg_sh.at[pl.ds(_NS * _RPT, _N - _NS * _RPT)],
                    agg_out.at[c].at[pl.ds(_NS * _RPT, _N - _NS * _RPT)])

  @pl.when(jnp.logical_and(s == 0, c == 0))
  def _():
    pltpu.sync_copy(den_sh, den_out0)

  @pl.when(jnp.logical_and(s == 0, c == 1))
  def _():
    pltpu.sync_copy(den_sh, den_out1)


def _sc_agg(h2, src, dst, ef):
  mesh = plsc.VectorSubcoreMesh(
      core_axis_name="c", subcore_axis_name="s",
      num_cores=_NC, num_subcores=_NS)
  fn = pl.kernel(
      _sc_agg_body,
      out_type=[
          jax.ShapeDtypeStruct((_NC, _N, _D), jnp.float32),
          jax.ShapeDtypeStruct((_N,), jnp.float32),
          jax.ShapeDtypeStruct((_N,), jnp.float32),
      ],
      mesh=mesh,
      scratch_types=[
          pltpu.VMEM((3, _CH), jnp.int32),
          pltpu.VMEM((3, _CH), jnp.int32),
          pltpu.VMEM((3, _CH), jnp.float32),
          pltpu.VMEM((2, _CH, _D), jnp.float32),
          pltpu.VMEM((2, _CH, _D), jnp.float32),
          pltpu.VMEM((_ZLEN,), jnp.float32),
          pltpu.VMEM_SHARED((_N, _D), jnp.float32),
          pltpu.VMEM_SHARED((_N,), jnp.float32),
          pltpu.SemaphoreType.DMA,
          pltpu.SemaphoreType.DMA,
          pltpu.SemaphoreType.DMA,
          pltpu.SemaphoreType.DMA,
      ],
      name="sc_edge_aggregate",
  )
  return fn(h2, src, dst, ef)


def _tc_pre_body(x_ref, w1_ref, b1_ref, w2_ref, b2_ref, h2_ref):
  x = x_ref[...]
  h = lax.dot_general(x, w1_ref[...], (((1,), (1,)), ((), ())),
                      preferred_element_type=jnp.float32)
  h = jnp.maximum(h + b1_ref[...], 0.0)
  h2_ref[...] = lax.dot_general(h, w2_ref[...], (((1,), (1,)), ((), ())),
                                preferred_element_type=jnp.float32) + b2_ref[...]


_RB = 2000


def _tc_pre(x, w1, b1, w2, b2):
  nb = _N // _RB
  return pl.pallas_call(
      _tc_pre_body,
      grid=(nb,),
      in_specs=[
          pl.BlockSpec((_RB, _D), lambda i: (i, 0)),
          pl.BlockSpec((_H, _D), lambda i: (0, 0)),
          pl.BlockSpec((1, _H), lambda i: (0, 0)),
          pl.BlockSpec((_D, _H), lambda i: (0, 0)),
          pl.BlockSpec((1, _D), lambda i: (0, 0)),
      ],
      out_specs=pl.BlockSpec((_RB, _D), lambda i: (i, 0)),
      out_shape=jax.ShapeDtypeStruct((_N, _D), jnp.float32),
  )(x, w1, b1.reshape(1, _H), w2, b2.reshape(1, _D))


def _tc_post_body(x_ref, w1_ref, b1_ref, w2_ref, b2_ref, agg_ref, den0_ref,
                  den1_ref, o_ref):
  x = x_ref[...]
  h = lax.dot_general(x, w1_ref[...], (((1,), (1,)), ((), ())),
                      preferred_element_type=jnp.float32)
  h = jnp.maximum(h + b1_ref[...], 0.0)
  h1 = lax.dot_general(h, w2_ref[...], (((1,), (1,)), ((), ())),
                       preferred_element_type=jnp.float32) + b2_ref[...]
  agg = agg_ref[0] + agg_ref[1]
  den = den0_ref[...] + den1_ref[...] + 1e-07
  inter = h1 + agg / den
  mean = jnp.mean(inter, axis=1, keepdims=True)
  cen = inter - mean
  var = jnp.mean(cen * cen, axis=1, keepdims=True)
  normed = cen * lax.rsqrt(var + 1e-05)
  o_ref[...] = x + jnp.maximum(normed, 0.0)


def _tc_post(x, w1, b1, w2, b2, agg_p, den0, den1):
  nb = _N // _RB
  return pl.pallas_call(
      _tc_post_body,
      grid=(nb,),
      in_specs=[
          pl.BlockSpec((_RB, _D), lambda i: (i, 0)),
          pl.BlockSpec((_H, _D), lambda i: (0, 0)),
          pl.BlockSpec((1, _H), lambda i: (0, 0)),
          pl.BlockSpec((_D, _H), lambda i: (0, 0)),
          pl.BlockSpec((1, _D), lambda i: (0, 0)),
          pl.BlockSpec((_NC, _RB, _D), lambda i: (0, i, 0)),
          pl.BlockSpec((_RB, 1), lambda i: (i, 0)),
          pl.BlockSpec((_RB, 1), lambda i: (i, 0)),
      ],
      out_specs=pl.BlockSpec((_RB, _D), lambda i: (i, 0)),
      out_shape=jax.ShapeDtypeStruct((_N, _D), jnp.float32),
  )(x, w1, b1.reshape(1, _H), w2, b2.reshape(1, _D), agg_p,
    den0.reshape(_N, 1), den1.reshape(_N, 1))


def kernel(node_features, edge_index, edge_features,
           W1a, b1a, W2a, b2a, W1b, b1b, W2b, b2b):
  src = edge_index[0].astype(jnp.int32)
  dst = edge_index[1].astype(jnp.int32)
  h2 = _tc_pre(node_features, W1b, b1b, W2b, b2b)
  agg_p, den0, den1 = _sc_agg(h2, src, dst, edge_features)
  return _tc_post(node_features, W1a, b1a, W2a, b2a, agg_p, den0, den1)


# speedup vs baseline: 2.8840x; 1.0359x over previous
import functools

import jax
import jax.numpy as jnp
from jax import lax
from jax.experimental import pallas as pl
from jax.experimental.pallas import tpu as pltpu
from jax.experimental.pallas import tpu_sc as plsc

_N = 10000
_E = 320000
_D = 128
_H = 128

_NC = 2
_NS = 16
_L = 16
_NW = _NC * _NS
_EPW = _E // _NW
_CH = 80
_NCHUNK = _EPW // _CH
_G = 1
_GB = _CH // _G
_RPT = 624
_ZLEN = 2000


def _sc_agg_body(h2, srcb, dstb, efb, agg_out, den_out0, den_out1,
                 src_v, dst_v, sig_v, rows0, rows1, zv,
                 agg_sh, den_sh, gsem, ssem, isem0, isem1):
  c = lax.axis_index("c")
  s = lax.axis_index("s")
  wid = s * _NC + c

  zero16 = jnp.zeros((_L,), jnp.float32)

  def _zr(i, carry):
    for k in range(_D // _L):
      rows0[0, i, pl.ds(k * _L, _L)] = zero16
    return carry
  lax.fori_loop(0, _CH, _zr, 0)

  def _zz(i, carry):
    zv[pl.ds(i * _L, _L)] = zero16
    return carry
  lax.fori_loop(0, _ZLEN // _L, _zz, 0)

  for z in range(_RPT // _CH):
    pltpu.sync_copy(rows0.at[0], agg_sh.at[pl.ds(s * _RPT + z * _CH, _CH)])
  pltpu.sync_copy(rows0.at[0].at[pl.ds(0, _RPT % _CH)],
                  agg_sh.at[pl.ds(s * _RPT + (_RPT // _CH) * _CH,
                                  _RPT % _CH)])

  @pl.when(s == _NS - 1)
  def _():
    pltpu.sync_copy(rows0.at[0].at[pl.ds(0, _N - _NS * _RPT)],
                    agg_sh.at[pl.ds(_NS * _RPT, _N - _NS * _RPT)])

  @pl.when(s == 0)
  def _():
    for i in range(_N // _ZLEN):
      pltpu.sync_copy(zv, den_sh.at[pl.ds(i * _ZLEN, _ZLEN)])

  plsc.subcore_barrier()

  ebase0 = wid * _EPW

  def _stage_fire(slot, ch, sem):
    eb = ebase0 + ch * _CH
    pltpu.async_copy(srcb.at[pl.ds(eb, _CH)], src_v.at[slot], sem)
    pltpu.async_copy(dstb.at[pl.ds(eb, _CH)], dst_v.at[slot], sem)
    pltpu.async_copy(efb.at[pl.ds(eb, _CH)], sig_v.at[slot], sem)

  def _stage_drain(sem):
    pltpu.make_async_copy(srcb.at[pl.ds(0, _CH)], src_v.at[0], sem).wait()
    pltpu.make_async_copy(dstb.at[pl.ds(0, _CH)], dst_v.at[0], sem).wait()
    pltpu.make_async_copy(efb.at[pl.ds(0, _CH)], sig_v.at[0], sem).wait()

  def _gather_fire(par, slot):
    pltpu.async_copy(h2.at[dst_v.at[slot]], rows0.at[par], gsem)
    pltpu.async_copy(h2.at[src_v.at[slot]], rows1.at[par], gsem)

  def _gather_drain():
    pltpu.make_async_copy(h2.at[pl.ds(0, _CH)], rows0.at[0], gsem).wait()
    pltpu.make_async_copy(h2.at[pl.ds(0, _CH)], rows1.at[0], gsem).wait()

  def _scatter_fire(par, slot):
    pltpu.async_copy(rows0.at[par], agg_sh.at[src_v.at[slot]], ssem,
                     add=True)
    pltpu.async_copy(rows1.at[par], agg_sh.at[dst_v.at[slot]], ssem,
                     add=True)
    pltpu.async_copy(sig_v.at[slot], den_sh.at[src_v.at[slot]], ssem,
                     add=True)
    pltpu.async_copy(sig_v.at[slot], den_sh.at[dst_v.at[slot]], ssem,
                     add=True)

  def _scatter_drain():
    pltpu.make_async_copy(rows0.at[0], agg_sh.at[pl.ds(0, _CH)],
                          ssem).wait()
    pltpu.make_async_copy(rows1.at[0], agg_sh.at[pl.ds(0, _CH)],
                          ssem).wait()
    pltpu.make_async_copy(sig_v.at[0], den_sh.at[pl.ds(0, _CH)],
                          ssem).wait()
    pltpu.make_async_copy(sig_v.at[0], den_sh.at[pl.ds(0, _CH)],
                          ssem).wait()

  bdnums = lax.GatherDimensionNumbers(
      offset_dims=(), collapsed_slice_dims=(0,), start_index_map=(0,))

  def _process(par, slot):
    for t in range(_CH // _L):
      sl = (slot, pl.ds(t * _L, _L))
      e = sig_v[sl]
      sig_v[sl] = 1.0 / (1.0 + jnp.exp(-e))

    @plsc.parallel_loop(0, _CH, 1, unroll=8)
    def _srow(row):
      t = lax.shift_right_logical(row, 4)
      r = jnp.bitwise_and(row, _L - 1)
      sv = sig_v[slot, pl.ds(t * _L, _L)]
      b = lax.gather(sv, jnp.full((_L, 1), r, jnp.int32), bdnums, (1,),
                     mode=lax.GatherScatterMode.PROMISE_IN_BOUNDS)
      for kk in range(_D // _L):
        sl = (par, row, pl.ds(kk * _L, _L))
        rows0[sl] = rows0[sl] * b
        rows1[sl] = rows1[sl] * b

  pltpu.sync_copy(srcb.at[pl.ds(ebase0, _CH)], src_v.at[0])
  pltpu.sync_copy(dstb.at[pl.ds(ebase0, _CH)], dst_v.at[0])
  pltpu.sync_copy(efb.at[pl.ds(ebase0, _CH)], sig_v.at[0])
  _stage_fire(1, 1, isem0)
  _gather_fire(0, 0)

  _gather_drain()
  _stage_drain(isem0)
  _gather_fire(1, 1)
  _stage_fire(2, 2, isem0)
  _process(0, 0)
  _scatter_fire(0, 0)

  def _chunk(k, carry):
    par = lax.rem(k, 2)
    par1 = lax.rem(k + 1, 2)
    slot = lax.rem(k, 3)
    slot1 = lax.rem(k + 1, 3)
    slot2 = lax.rem(k + 2, 3)
    _gather_drain()
    _scatter_drain()
    _stage_drain(isem0)
    _gather_fire(par1, slot1)
    _stage_fire(slot2, k + 2, isem0)
    _process(par, slot)
    _scatter_fire(par, slot)
    return carry

  lax.fori_loop(1, _NCHUNK - 2, _chunk, 0)

  kA = _NCHUNK - 2
  _gather_drain()
  _scatter_drain()
  _stage_drain(isem0)
  _gather_fire((kA + 1) % 2, (kA + 1) % 3)
  _process(kA % 2, kA % 3)
  _scatter_fire(kA % 2, kA % 3)

  kB = _NCHUNK - 1
  _gather_drain()
  _scatter_drain()
  _process(kB % 2, kB % 3)
  _scatter_fire(kB % 2, kB % 3)
  _scatter_drain()

  plsc.subcore_barrier()

  pltpu.sync_copy(agg_sh.at[pl.ds(s * _RPT, _RPT)],
                  agg_out.at[c].at[pl.ds(s * _RPT, _RPT)])

  @pl.when(s == _NS - 1)
  def _():
    pltpu.sync_copy(agg_sh.at[pl.ds(_NS * _RPT, _N - _NS * _RPT)],
                    agg_out.at[c].at[pl.ds(_NS * _RPT, _N - _NS * _RPT)])

  @pl.when(jnp.logical_and(s == 0, c == 0))
  def _():
    pltpu.sync_copy(den_sh, den_out0)

  @pl.when(jnp.logical_and(s == 0, c == 1))
  def _():
    pltpu.sync_copy(den_sh, den_out1)


def _sc_agg(h2, src, dst, ef):
  mesh = plsc.VectorSubcoreMesh(
      core_axis_name="c", subcore_axis_name="s",
      num_cores=_NC, num_subcores=_NS)
  fn = pl.kernel(
      _sc_agg_body,
      out_type=[
          jax.ShapeDtypeStruct((_NC, _N, _D), jnp.float32),
          jax.ShapeDtypeStruct((_N,), jnp.float32),
          jax.ShapeDtypeStruct((_N,), jnp.float32),
      ],
      mesh=mesh,
      scratch_types=[
          pltpu.VMEM((3, _CH), jnp.int32),
          pltpu.VMEM((3, _CH), jnp.int32),
          pltpu.VMEM((3, _CH), jnp.float32),
          pltpu.VMEM((2, _CH, _D), jnp.float32),
          pltpu.VMEM((2, _CH, _D), jnp.float32),
          pltpu.VMEM((_ZLEN,), jnp.float32),
          pltpu.VMEM_SHARED((_N, _D), jnp.float32),
          pltpu.VMEM_SHARED((_N,), jnp.float32),
          pltpu.SemaphoreType.DMA,
          pltpu.SemaphoreType.DMA,
          pltpu.SemaphoreType.DMA,
          pltpu.SemaphoreType.DMA,
      ],
      name="sc_edge_aggregate",
  )
  return fn(h2, src, dst, ef)


def _tc_pre_body(x_ref, w1_ref, b1_ref, w2_ref, b2_ref, h2_ref):
  x = x_ref[...]
  h = lax.dot_general(x, w1_ref[...], (((1,), (1,)), ((), ())),
                      preferred_element_type=jnp.float32)
  h = jnp.maximum(h + b1_ref[...], 0.0)
  h2_ref[...] = lax.dot_general(h, w2_ref[...], (((1,), (1,)), ((), ())),
                                preferred_element_type=jnp.float32) + b2_ref[...]


_RB = 2000


def _tc_pre(x, w1, b1, w2, b2):
  nb = _N // _RB
  return pl.pallas_call(
      _tc_pre_body,
      grid=(nb,),
      in_specs=[
          pl.BlockSpec((_RB, _D), lambda i: (i, 0)),
          pl.BlockSpec((_H, _D), lambda i: (0, 0)),
          pl.BlockSpec((1, _H), lambda i: (0, 0)),
          pl.BlockSpec((_D, _H), lambda i: (0, 0)),
          pl.BlockSpec((1, _D), lambda i: (0, 0)),
      ],
      out_specs=pl.BlockSpec((_RB, _D), lambda i: (i, 0)),
      out_shape=jax.ShapeDtypeStruct((_N, _D), jnp.float32),
  )(x, w1, b1.reshape(1, _H), w2, b2.reshape(1, _D))


def _tc_post_body(x_ref, w1_ref, b1_ref, w2_ref, b2_ref, agg_ref, den0_ref,
                  den1_ref, o_ref):
  x = x_ref[...]
  h = lax.dot_general(x, w1_ref[...], (((1,), (1,)), ((), ())),
                      preferred_element_type=jnp.float32)
  h = jnp.maximum(h + b1_ref[...], 0.0)
  h1 = lax.dot_general(h, w2_ref[...], (((1,), (1,)), ((), ())),
                       preferred_element_type=jnp.float32) + b2_ref[...]
  agg = agg_ref[0] + agg_ref[1]
  den = den0_ref[...] + den1_ref[...] + 1e-07
  inter = h1 + agg / den
  mean = jnp.mean(inter, axis=1, keepdims=True)
  cen = inter - mean
  var = jnp.mean(cen * cen, axis=1, keepdims=True)
  normed = cen * lax.rsqrt(var + 1e-05)
  o_ref[...] = x + jnp.maximum(normed, 0.0)


def _tc_post(x, w1, b1, w2, b2, agg_p, den0, den1):
  nb = _N // _RB
  return pl.pallas_call(
      _tc_post_body,
      grid=(nb,),
      in_specs=[
          pl.BlockSpec((_RB, _D), lambda i: (i, 0)),
          pl.BlockSpec((_H, _D), lambda i: (0, 0)),
          pl.BlockSpec((1, _H), lambda i: (0, 0)),
          pl.BlockSpec((_D, _H), lambda i: (0, 0)),
          pl.BlockSpec((1, _D), lambda i: (0, 0)),
          pl.BlockSpec((_NC, _RB, _D), lambda i: (0, i, 0)),
          pl.BlockSpec((_RB, 1), lambda i: (i, 0)),
          pl.BlockSpec((_RB, 1), lambda i: (i, 0)),
      ],
      out_specs=pl.BlockSpec((_RB, _D), lambda i: (i, 0)),
      out_shape=jax.ShapeDtypeStruct((_N, _D), jnp.float32),
  )(x, w1, b1.reshape(1, _H), w2, b2.reshape(1, _D), agg_p,
    den0.reshape(_N, 1), den1.reshape(_N, 1))


def kernel(node_features, edge_index, edge_features,
           W1a, b1a, W2a, b2a, W1b, b1b, W2b, b2b):
  src = edge_index[0].astype(jnp.int32)
  dst = edge_index[1].astype(jnp.int32)
  h2 = _tc_pre(node_features, W1b, b1b, W2b, b2b)
  agg_p, den0, den1 = _sc_agg(h2, src, dst, edge_features)
  return _tc_post(node_features, W1a, b1a, W2a, b2a, agg_p, den0, den1)


# speedup vs baseline: 2.9013x; 1.0060x over previous
import functools

import jax
import jax.numpy as jnp
from jax import lax
from jax.experimental import pallas as pl
from jax.experimental.pallas import tpu as pltpu
from jax.experimental.pallas import tpu_sc as plsc

_N = 10000
_E = 320000
_D = 128
_H = 128

_NC = 2
_NS = 16
_L = 16
_NW = _NC * _NS
_EPW = _E // _NW
_CH = 80
_NCHUNK = _EPW // _CH
_G = 1
_GB = _CH // _G
_RPT = 624
_ZLEN = 2000


def _sc_agg_body(h2, srcb, dstb, efb, agg_out, den_out0, den_out1,
                 src_v, dst_v, sig_v, rows0, rows1, zv,
                 agg_sh, den_sh, gsem, ssem, isem0, isem1):
  c = lax.axis_index("c")
  s = lax.axis_index("s")
  wid = s * _NC + c

  zero16 = jnp.zeros((_L,), jnp.float32)

  def _zr(i, carry):
    for k in range(_D // _L):
      rows0[0, i, pl.ds(k * _L, _L)] = zero16
    return carry
  lax.fori_loop(0, _CH, _zr, 0)

  def _zz(i, carry):
    zv[pl.ds(i * _L, _L)] = zero16
    return carry
  lax.fori_loop(0, _ZLEN // _L, _zz, 0)

  for z in range(_RPT // _CH):
    pltpu.sync_copy(rows0.at[0], agg_sh.at[pl.ds(s * _RPT + z * _CH, _CH)])
  pltpu.sync_copy(rows0.at[0].at[pl.ds(0, _RPT % _CH)],
                  agg_sh.at[pl.ds(s * _RPT + (_RPT // _CH) * _CH,
                                  _RPT % _CH)])

  @pl.when(s == _NS - 1)
  def _():
    pltpu.sync_copy(rows0.at[0].at[pl.ds(0, _N - _NS * _RPT)],
                    agg_sh.at[pl.ds(_NS * _RPT, _N - _NS * _RPT)])

  @pl.when(s == 0)
  def _():
    for i in range(_N // _ZLEN):
      pltpu.sync_copy(zv, den_sh.at[pl.ds(i * _ZLEN, _ZLEN)])

  plsc.subcore_barrier()

  ebase0 = wid * _EPW

  def _stage_fire(slot, ch, sem):
    eb = ebase0 + ch * _CH
    pltpu.async_copy(srcb.at[pl.ds(eb, _CH)], src_v.at[slot], sem)
    pltpu.async_copy(dstb.at[pl.ds(eb, _CH)], dst_v.at[slot], sem)
    pltpu.async_copy(efb.at[pl.ds(eb, _CH)], sig_v.at[slot], sem)

  def _stage_drain(sem):
    pltpu.make_async_copy(srcb.at[pl.ds(0, _CH)], src_v.at[0], sem).wait()
    pltpu.make_async_copy(dstb.at[pl.ds(0, _CH)], dst_v.at[0], sem).wait()
    pltpu.make_async_copy(efb.at[pl.ds(0, _CH)], sig_v.at[0], sem).wait()

  def _gather_fire(par, slot):
    for hh in range(2):
      hs = pl.ds(hh * (_CH // 2), _CH // 2)
      pltpu.async_copy(h2.at[dst_v.at[slot].at[hs]],
                       rows0.at[par].at[hs], gsem)
      pltpu.async_copy(h2.at[src_v.at[slot].at[hs]],
                       rows1.at[par].at[hs], gsem)

  def _gather_drain():
    for hh in range(2):
      hs = pl.ds(hh * (_CH // 2), _CH // 2)
      pltpu.make_async_copy(h2.at[pl.ds(0, _CH // 2)],
                            rows0.at[0].at[hs], gsem).wait()
      pltpu.make_async_copy(h2.at[pl.ds(0, _CH // 2)],
                            rows1.at[0].at[hs], gsem).wait()

  def _scatter_fire(par, slot):
    pltpu.async_copy(rows0.at[par], agg_sh.at[src_v.at[slot]], ssem,
                     add=True)
    pltpu.async_copy(rows1.at[par], agg_sh.at[dst_v.at[slot]], ssem,
                     add=True)
    pltpu.async_copy(sig_v.at[slot], den_sh.at[src_v.at[slot]], ssem,
                     add=True)
    pltpu.async_copy(sig_v.at[slot], den_sh.at[dst_v.at[slot]], ssem,
                     add=True)

  def _scatter_drain():
    pltpu.make_async_copy(rows0.at[0], agg_sh.at[pl.ds(0, _CH)],
                          ssem).wait()
    pltpu.make_async_copy(rows1.at[0], agg_sh.at[pl.ds(0, _CH)],
                          ssem).wait()
    pltpu.make_async_copy(sig_v.at[0], den_sh.at[pl.ds(0, _CH)],
                          ssem).wait()
    pltpu.make_async_copy(sig_v.at[0], den_sh.at[pl.ds(0, _CH)],
                          ssem).wait()

  bdnums = lax.GatherDimensionNumbers(
      offset_dims=(), collapsed_slice_dims=(0,), start_index_map=(0,))

  def _process(par, slot):
    for t in range(_CH // _L):
      sl = (slot, pl.ds(t * _L, _L))
      e = sig_v[sl]
      sig_v[sl] = 1.0 / (1.0 + jnp.exp(-e))

    @plsc.parallel_loop(0, _CH, 1, unroll=8)
    def _srow(row):
      t = lax.shift_right_logical(row, 4)
      r = jnp.bitwise_and(row, _L - 1)
      sv = sig_v[slot, pl.ds(t * _L, _L)]
      b = lax.gather(sv, jnp.full((_L, 1), r, jnp.int32), bdnums, (1,),
                     mode=lax.GatherScatterMode.PROMISE_IN_BOUNDS)
      for kk in range(_D // _L):
        sl = (par, row, pl.ds(kk * _L, _L))
        rows0[sl] = rows0[sl] * b
        rows1[sl] = rows1[sl] * b

  pltpu.sync_copy(srcb.at[pl.ds(ebase0, _CH)], src_v.at[0])
  pltpu.sync_copy(dstb.at[pl.ds(ebase0, _CH)], dst_v.at[0])
  pltpu.sync_copy(efb.at[pl.ds(ebase0, _CH)], sig_v.at[0])
  _stage_fire(1, 1, isem0)
  _gather_fire(0, 0)

  _gather_drain()
  _stage_drain(isem0)
  _gather_fire(1, 1)
  _stage_fire(2, 2, isem0)
  _process(0, 0)
  _scatter_fire(0, 0)

  def _chunk(k, carry):
    par = lax.rem(k, 2)
    par1 = lax.rem(k + 1, 2)
    slot = lax.rem(k, 3)
    slot1 = lax.rem(k + 1, 3)
    slot2 = lax.rem(k + 2, 3)
    _gather_drain()
    _scatter_drain()
    _stage_drain(isem0)
    _gather_fire(par1, slot1)
    _stage_fire(slot2, k + 2, isem0)
    _process(par, slot)
    _scatter_fire(par, slot)
    return carry

  lax.fori_loop(1, _NCHUNK - 2, _chunk, 0)

  kA = _NCHUNK - 2
  _gather_drain()
  _scatter_drain()
  _stage_drain(isem0)
  _gather_fire((kA + 1) % 2, (kA + 1) % 3)
  _process(kA % 2, kA % 3)
  _scatter_fire(kA % 2, kA % 3)

  kB = _NCHUNK - 1
  _gather_drain()
  _scatter_drain()
  _process(kB % 2, kB % 3)
  _scatter_fire(kB % 2, kB % 3)
  _scatter_drain()

  plsc.subcore_barrier()

  pltpu.sync_copy(agg_sh.at[pl.ds(s * _RPT, _RPT)],
                  agg_out.at[c].at[pl.ds(s * _RPT, _RPT)])

  @pl.when(s == _NS - 1)
  def _():
    pltpu.sync_copy(agg_sh.at[pl.ds(_NS * _RPT, _N - _NS * _RPT)],
                    agg_out.at[c].at[pl.ds(_NS * _RPT, _N - _NS * _RPT)])

  @pl.when(jnp.logical_and(s == 0, c == 0))
  def _():
    pltpu.sync_copy(den_sh, den_out0)

  @pl.when(jnp.logical_and(s == 0, c == 1))
  def _():
    pltpu.sync_copy(den_sh, den_out1)


def _sc_agg(h2, src, dst, ef):
  mesh = plsc.VectorSubcoreMesh(
      core_axis_name="c", subcore_axis_name="s",
      num_cores=_NC, num_subcores=_NS)
  fn = pl.kernel(
      _sc_agg_body,
      out_type=[
          jax.ShapeDtypeStruct((_NC, _N, _D), jnp.float32),
          jax.ShapeDtypeStruct((_N,), jnp.float32),
          jax.ShapeDtypeStruct((_N,), jnp.float32),
      ],
      mesh=mesh,
      scratch_types=[
          pltpu.VMEM((3, _CH), jnp.int32),
          pltpu.VMEM((3, _CH), jnp.int32),
          pltpu.VMEM((3, _CH), jnp.float32),
          pltpu.VMEM((2, _CH, _D), jnp.float32),
          pltpu.VMEM((2, _CH, _D), jnp.float32),
          pltpu.VMEM((_ZLEN,), jnp.float32),
          pltpu.VMEM_SHARED((_N, _D), jnp.float32),
          pltpu.VMEM_SHARED((_N,), jnp.float32),
          pltpu.SemaphoreType.DMA,
          pltpu.SemaphoreType.DMA,
          pltpu.SemaphoreType.DMA,
          pltpu.SemaphoreType.DMA,
      ],
      name="sc_edge_aggregate",
  )
  return fn(h2, src, dst, ef)


def _tc_pre_body(x_ref, w1_ref, b1_ref, w2_ref, b2_ref, h2_ref):
  x = x_ref[...]
  h = lax.dot_general(x, w1_ref[...], (((1,), (1,)), ((), ())),
                      preferred_element_type=jnp.float32)
  h = jnp.maximum(h + b1_ref[...], 0.0)
  h2_ref[...] = lax.dot_general(h, w2_ref[...], (((1,), (1,)), ((), ())),
                                preferred_element_type=jnp.float32) + b2_ref[...]


_RB = 2000


def _tc_pre(x, w1, b1, w2, b2):
  nb = _N // _RB
  return pl.pallas_call(
      _tc_pre_body,
      grid=(nb,),
      in_specs=[
          pl.BlockSpec((_RB, _D), lambda i: (i, 0)),
          pl.BlockSpec((_H, _D), lambda i: (0, 0)),
          pl.BlockSpec((1, _H), lambda i: (0, 0)),
          pl.BlockSpec((_D, _H), lambda i: (0, 0)),
          pl.BlockSpec((1, _D), lambda i: (0, 0)),
      ],
      out_specs=pl.BlockSpec((_RB, _D), lambda i: (i, 0)),
      out_shape=jax.ShapeDtypeStruct((_N, _D), jnp.float32),
  )(x, w1, b1.reshape(1, _H), w2, b2.reshape(1, _D))


def _tc_post_body(x_ref, w1_ref, b1_ref, w2_ref, b2_ref, agg_ref, den0_ref,
                  den1_ref, o_ref):
  x = x_ref[...]
  h = lax.dot_general(x, w1_ref[...], (((1,), (1,)), ((), ())),
                      preferred_element_type=jnp.float32)
  h = jnp.maximum(h + b1_ref[...], 0.0)
  h1 = lax.dot_general(h, w2_ref[...], (((1,), (1,)), ((), ())),
                       preferred_element_type=jnp.float32) + b2_ref[...]
  agg = agg_ref[0] + agg_ref[1]
  den = den0_ref[...] + den1_ref[...] + 1e-07
  inter = h1 + agg / den
  mean = jnp.mean(inter, axis=1, keepdims=True)
  cen = inter - mean
  var = jnp.mean(cen * cen, axis=1, keepdims=True)
  normed = cen * lax.rsqrt(var + 1e-05)
  o_ref[...] = x + jnp.maximum(normed, 0.0)


def _tc_post(x, w1, b1, w2, b2, agg_p, den0, den1):
  nb = _N // _RB
  return pl.pallas_call(
      _tc_post_body,
      grid=(nb,),
      in_specs=[
          pl.BlockSpec((_RB, _D), lambda i: (i, 0)),
          pl.BlockSpec((_H, _D), lambda i: (0, 0)),
          pl.BlockSpec((1, _H), lambda i: (0, 0)),
          pl.BlockSpec((_D, _H), lambda i: (0, 0)),
          pl.BlockSpec((1, _D), lambda i: (0, 0)),
          pl.BlockSpec((_NC, _RB, _D), lambda i: (0, i, 0)),
          pl.BlockSpec((_RB, 1), lambda i: (i, 0)),
          pl.BlockSpec((_RB, 1), lambda i: (i, 0)),
      ],
      out_specs=pl.BlockSpec((_RB, _D), lambda i: (i, 0)),
      out_shape=jax.ShapeDtypeStruct((_N, _D), jnp.float32),
  )(x, w1, b1.reshape(1, _H), w2, b2.reshape(1, _D), agg_p,
    den0.reshape(_N, 1), den1.reshape(_N, 1))


def kernel(node_features, edge_index, edge_features,
           W1a, b1a, W2a, b2a, W1b, b1b, W2b, b2b):
  src = edge_index[0].astype(jnp.int32)
  dst = edge_index[1].astype(jnp.int32)
  h2 = _tc_pre(node_features, W1b, b1b, W2b, b2b)
  agg_p, den0, den1 = _sc_agg(h2, src, dst, edge_features)
  return _tc_post(node_features, W1a, b1a, W2a, b2a, agg_p, den0, den1)
